# baseline (speedup 1.0000x reference)
import functools

import jax
import jax.numpy as jnp
from jax import lax
from jax.experimental import pallas as pl
from jax.experimental.pallas import tpu as pltpu
from jax.experimental.pallas import tpu_sc as plsc

BATCH = 4096
HIDDEN = 1024
VOCAB = 1001
NUM_CORES = 2
NUM_SUBCORES = 16
NUM_WORKERS = NUM_CORES * NUM_SUBCORES
B_PER_W = BATCH // NUM_WORKERS


@functools.partial(
    pl.kernel,
    mesh=plsc.VectorSubcoreMesh(core_axis_name="c", subcore_axis_name="s"),
    out_type=jax.ShapeDtypeStruct((BATCH * HIDDEN,), jnp.float32),
    scratch_types=[
        pltpu.VMEM((B_PER_W,), jnp.int32),
        pltpu.SemaphoreType.DMA,
    ],
)
def _gather_kernel(table_hbm, idx_hbm, out_hbm, idx_v, sem):
    wid = lax.axis_index("s") * NUM_CORES + lax.axis_index("c")
    base = wid * B_PER_W
    pltpu.sync_copy(idx_hbm.at[pl.ds(base, B_PER_W)], idx_v)
    for k in range(B_PER_W // 16):
        vec = idx_v[pl.ds(k * 16, 16)]
        for j in range(16):
            i = k * 16 + j
            lab = vec[j]
            pltpu.async_copy(
                table_hbm.at[pl.ds(pl.multiple_of(lab * HIDDEN, 8), HIDDEN)],
                out_hbm.at[pl.ds((base + i) * HIDDEN, HIDDEN)],
                sem,
            )
    for i in range(B_PER_W):
        pltpu.make_async_copy(
            table_hbm.at[pl.ds(0, HIDDEN)],
            out_hbm.at[pl.ds(base * HIDDEN, HIDDEN)],
            sem,
        ).wait()


def kernel(labels, embedding_table):
    flat = _gather_kernel(embedding_table.reshape(-1), labels.astype(jnp.int32))
    return flat.reshape(BATCH, HIDDEN)


# speedup vs baseline: 16.7647x; 16.7647x over previous
import functools

import jax
import jax.numpy as jnp
from jax import lax
from jax.experimental import pallas as pl
from jax.experimental.pallas import tpu as pltpu
from jax.experimental.pallas import tpu_sc as plsc

BATCH = 4096
HIDDEN = 1024
NUM_CORES = 2
NUM_SUBCORES = 16
NUM_WORKERS = NUM_CORES * NUM_SUBCORES
B_PER_W = BATCH // NUM_WORKERS
CHUNKS = (8, 8, 16, 32, 32, 16, 8, 8)
OFFS = tuple(sum(CHUNKS[:i]) for i in range(len(CHUNKS)))
NCHUNK = len(CHUNKS)
MAXC = max(CHUNKS)
NBUF = 3


@functools.partial(
    pl.kernel,
    mesh=plsc.VectorSubcoreMesh(core_axis_name="c", subcore_axis_name="s"),
    out_type=jax.ShapeDtypeStruct((BATCH, HIDDEN), jnp.float32),
    scratch_types=[
        pltpu.VMEM((B_PER_W,), jnp.int32),
        pltpu.VMEM((NBUF, MAXC, HIDDEN), jnp.float32),
        pltpu.SemaphoreType.DMA,
        pltpu.SemaphoreType.DMA,
    ],
)
def _gather_kernel(table_hbm, idx_hbm, out_hbm, idx_v, rows_v, gsem, osem):
    wid = lax.axis_index("s") * NUM_CORES + lax.axis_index("c")
    base = wid * B_PER_W

    def gather_copy(c):
        return pltpu.make_async_copy(
            table_hbm.at[idx_v.at[pl.ds(OFFS[c], CHUNKS[c])]],
            rows_v.at[c % NBUF].at[pl.ds(0, CHUNKS[c])],
            gsem,
        )

    def out_copy(c):
        return pltpu.make_async_copy(
            rows_v.at[c % NBUF].at[pl.ds(0, CHUNKS[c])],
            out_hbm.at[pl.ds(base + OFFS[c], CHUNKS[c])],
            osem,
        )

    pltpu.sync_copy(idx_hbm.at[pl.ds(base, B_PER_W)], idx_v)

    for c in range(min(NBUF - 1, NCHUNK)):
        gather_copy(c).start()
    for c in range(NCHUNK):
        gather_copy(c).wait()
        out_copy(c).start()
        nxt = c + NBUF - 1
        if nxt < NCHUNK:
            drain = nxt - NBUF
            if drain >= 0:
                out_copy(drain).wait()
            gather_copy(nxt).start()
    for c in range(max(NCHUNK - NBUF, 0), NCHUNK):
        out_copy(c).wait()


def kernel(labels, embedding_table):
    return _gather_kernel(embedding_table, labels.astype(jnp.int32))
